# baseline (speedup 1.0000x reference)
import functools

import jax
import jax.numpy as jnp
from jax import lax
from jax.experimental import pallas as pl
from jax.experimental.pallas import tpu as pltpu
from jax.experimental.pallas import tpu_sc as plsc

N = 10000
D = 128
E = 320000
OUT = 10

NCORES = 2
NSUB = 16
NW = NCORES * NSUB
K = 64
CHUNKS = 160
NBUF = 4
EPW = CHUNKS * K
E_PAD = EPW * NW
JUNK = 112
R = N + JUNK
RPS = R // NSUB
KV = K // 16
ZB = 32


def _segment_sum_partials(h, packed_hbm):
    @functools.partial(
        pl.kernel,
        out_type=jax.ShapeDtypeStruct((NCORES, N, D), jnp.float32),
        mesh=plsc.VectorSubcoreMesh(core_axis_name="c", subcore_axis_name="s"),
        scratch_types=(
            [pltpu.VMEM_SHARED((R, D), jnp.float32)]
            + [pltpu.VMEM((CHUNKS * K,), jnp.int32)]
            + [pltpu.VMEM((K,), jnp.int32) for _ in range(2 * NBUF)]
            + [pltpu.VMEM((K, D), jnp.float32) for _ in range(NBUF)]
            + [pltpu.VMEM((ZB, D), jnp.float32)]
            + [pltpu.SemaphoreType.DMA, pltpu.SemaphoreType.DMA]
        ),
    )
    def kern(h_hbm, packed, out_hbm, agg, idx_all, *rest):
        srcb = list(rest[0:NBUF])
        dstb = list(rest[NBUF:2 * NBUF])
        rowb = list(rest[2 * NBUF:3 * NBUF])
        zbuf = rest[3 * NBUF]
        semg = rest[3 * NBUF + 1]
        semz = rest[3 * NBUF + 2]
        cid = lax.axis_index("c")
        sid = lax.axis_index("s")
        wid = sid * NCORES + cid

        pltpu.async_copy(packed.at[wid], idx_all, semz)
        for r in range(ZB):
            for cc in range(D // 16):
                zbuf[r, pl.ds(cc * 16, 16)] = jnp.zeros((16,), jnp.float32)
        pltpu.make_async_copy(packed.at[wid], idx_all, semz).wait()

        def unpack(c, slot):
            for j in range(KV):
                v = idx_all[pl.ds(pl.multiple_of(c * K + j * 16, 16), 16)]
                srcb[slot][pl.ds(j * 16, 16)] = v >> 16
                dstb[slot][pl.ds(j * 16, 16)] = v & 0xFFFF

        def gather(c, slot):
            unpack(c, slot)
            pltpu.async_copy(h_hbm.at[srcb[slot]], rowb[slot], semg)

        def wait_gather(slot):
            pltpu.make_async_copy(h_hbm.at[pl.ds(0, K)], rowb[slot], semg).wait()

        for c in range(NBUF - 1):
            gather(c, c)

        base = sid * RPS
        for m in range(RPS // ZB):
            pltpu.async_copy(zbuf, agg.at[pl.ds(base + m * ZB, ZB)], semz)
        rem = RPS % ZB
        if rem:
            pltpu.async_copy(zbuf.at[pl.ds(0, rem)],
                             agg.at[pl.ds(base + RPS - rem, rem)], semz)
        for m in range(RPS // ZB):
            pltpu.make_async_copy(h_hbm.at[pl.ds(0, ZB)], zbuf, semz).wait()
        if rem:
            pltpu.make_async_copy(h_hbm.at[pl.ds(0, rem)],
                                  zbuf.at[pl.ds(0, rem)], semz).wait()
        plsc.subcore_barrier()

        def body(t, carry):
            c0 = NBUF * t
            for j in range(NBUF):
                wait_gather(j)

                @pl.when(c0 + j + NBUF - 1 < CHUNKS)
                def _():
                    gather(c0 + j + NBUF - 1, (j + NBUF - 1) % NBUF)

                pltpu.sync_copy(rowb[j], agg.at[dstb[j]], add=True)
            return carry

        lax.fori_loop(0, CHUNKS // NBUF, body, 0)
        plsc.subcore_barrier()

        @pl.when(sid < NSUB - 1)
        def _():
            pltpu.sync_copy(agg.at[pl.ds(sid * RPS, RPS)],
                            out_hbm.at[cid, pl.ds(sid * RPS, RPS)])

        @pl.when(sid == NSUB - 1)
        def _():
            tail = N - (NSUB - 1) * RPS
            pltpu.sync_copy(agg.at[pl.ds((NSUB - 1) * RPS, tail)],
                            out_hbm.at[cid, pl.ds((NSUB - 1) * RPS, tail)])

    return kern(h, packed_hbm)


BR = 2000


def _root_body(h_ref, wt_ref, b_ref, r_ref):
    r_ref[...] = (h_ref[...] + b_ref[...]
                  + jnp.dot(h_ref[...], wt_ref[...],
                            preferred_element_type=jnp.float32))


def _root_tc(h, W_root, b_rel):
    return pl.pallas_call(
        _root_body,
        grid=(N // BR,),
        in_specs=[
            pl.BlockSpec((BR, D), lambda i: (i, 0)),
            pl.BlockSpec((D, D), lambda i: (0, 0)),
            pl.BlockSpec((1, D), lambda i: (0, 0)),
        ],
        out_specs=pl.BlockSpec((BR, D), lambda i: (i, 0)),
        out_shape=jax.ShapeDtypeStruct((N, D), jnp.float32),
    )(h, W_root, b_rel.reshape(1, D))


def _fin_body(r_ref, p0_ref, p1_ref, wr_ref, o_ref):
    agg = p0_ref[0] + p1_ref[0]
    o = r_ref[...] + jnp.dot(agg, wr_ref[...],
                             preferred_element_type=jnp.float32)
    o_ref[...] = jnp.maximum(o, 0.0)


def _fin_tc(r, parts, W_rel):
    return pl.pallas_call(
        _fin_body,
        grid=(N // BR,),
        in_specs=[
            pl.BlockSpec((BR, D), lambda i: (i, 0)),
            pl.BlockSpec((1, BR, D), lambda i: (0, i, 0)),
            pl.BlockSpec((1, BR, D), lambda i: (1, i, 0)),
            pl.BlockSpec((D, D), lambda i: (0, 0)),
        ],
        out_specs=pl.BlockSpec((BR, D), lambda i: (i, 0)),
        out_shape=jax.ShapeDtypeStruct((N, D), jnp.float32),
    )(r, parts, parts, W_rel)


def _final_body(r_ref, p0_ref, p1_ref, wr_ref, wl_ref, bl_ref, o_ref):
    agg = p0_ref[0] + p1_ref[0]
    h5 = r_ref[...] + jnp.dot(agg, wr_ref[...],
                              preferred_element_type=jnp.float32)
    o_ref[...] = (jnp.dot(h5, wl_ref[...], preferred_element_type=jnp.float32)
                  + bl_ref[...])


def _final_tc(r, parts, W_rel, W_lin, b_lin):
    return pl.pallas_call(
        _final_body,
        grid=(N // BR,),
        in_specs=[
            pl.BlockSpec((BR, D), lambda i: (i, 0)),
            pl.BlockSpec((1, BR, D), lambda i: (0, i, 0)),
            pl.BlockSpec((1, BR, D), lambda i: (1, i, 0)),
            pl.BlockSpec((D, D), lambda i: (0, 0)),
            pl.BlockSpec((D, OUT), lambda i: (0, 0)),
            pl.BlockSpec((1, OUT), lambda i: (0, 0)),
        ],
        out_specs=pl.BlockSpec((BR, OUT), lambda i: (i, 0)),
        out_shape=jax.ShapeDtypeStruct((N, OUT), jnp.float32),
    )(r, parts, parts, W_rel, W_lin, b_lin.reshape(1, OUT))


def kernel(x, edge_index,
           W_rel1, b_rel1, W_root1, W_rel2, b_rel2, W_root2,
           W_rel3, b_rel3, W_root3, W_rel4, b_rel4, W_root4,
           W_rel5, b_rel5, W_root5, W_lin, b_lin):
    src, dst = edge_index[0], edge_index[1]
    pad = jnp.arange(E_PAD - E, dtype=jnp.int32)
    src_p = jnp.concatenate([src, (pad * 7919) % N])
    dst_p = jnp.concatenate([dst, N + pad % JUNK])
    packed = ((src_p << 16) | dst_p).reshape(NW, CHUNKS * K)
    layers = [(W_rel1, b_rel1, W_root1), (W_rel2, b_rel2, W_root2),
              (W_rel3, b_rel3, W_root3), (W_rel4, b_rel4, W_root4)]
    h = x
    for (Wr, br, Wt) in layers:
        parts = _segment_sum_partials(h, packed)
        r = _root_tc(h, Wt, br)
        h = _fin_tc(r, parts, Wr)
    parts = _segment_sum_partials(h, packed)
    r = _root_tc(h, W_root5, b_rel5)
    return _final_tc(r, parts, W_rel5, W_lin, b_lin)


# speedup vs baseline: 1.0098x; 1.0098x over previous
import functools

import jax
import jax.numpy as jnp
from jax import lax
from jax.experimental import pallas as pl
from jax.experimental.pallas import tpu as pltpu
from jax.experimental.pallas import tpu_sc as plsc

N = 10000
D = 128
E = 320000
OUT = 10

NCORES = 2
NSUB = 16
NW = NCORES * NSUB
K = 64
CHUNKS = 160
NBUF = 4
EPW = CHUNKS * K
E_PAD = EPW * NW
JUNK = 112
R = N + JUNK
RPS = R // NSUB
KV = K // 16
ZB = 32


def _segment_sum_partials(h, packed_hbm):
    @functools.partial(
        pl.kernel,
        out_type=jax.ShapeDtypeStruct((NCORES, N, D), jnp.float32),
        mesh=plsc.VectorSubcoreMesh(core_axis_name="c", subcore_axis_name="s"),
        scratch_types=(
            [pltpu.VMEM_SHARED((R, D), jnp.float32)]
            + [pltpu.VMEM((CHUNKS * K,), jnp.int32)]
            + [pltpu.VMEM((K,), jnp.int32) for _ in range(2 * NBUF)]
            + [pltpu.VMEM((K, D), jnp.float32) for _ in range(NBUF)]
            + [pltpu.VMEM((ZB, D), jnp.float32)]
            + [pltpu.SemaphoreType.DMA, pltpu.SemaphoreType.DMA]
        ),
    )
    def kern(h_hbm, packed, out_hbm, agg, idx_all, *rest):
        srcb = list(rest[0:NBUF])
        dstb = list(rest[NBUF:2 * NBUF])
        rowb = list(rest[2 * NBUF:3 * NBUF])
        zbuf = rest[3 * NBUF]
        semg = rest[3 * NBUF + 1]
        semz = rest[3 * NBUF + 2]
        cid = lax.axis_index("c")
        sid = lax.axis_index("s")
        wid = sid * NCORES + cid

        pltpu.async_copy(packed.at[wid], idx_all, semz)
        for r in range(ZB):
            for cc in range(D // 16):
                zbuf[r, pl.ds(cc * 16, 16)] = jnp.zeros((16,), jnp.float32)
        pltpu.make_async_copy(packed.at[wid], idx_all, semz).wait()

        def unpack(c, slot):
            for j in range(KV):
                v = idx_all[pl.ds(pl.multiple_of(c * K + j * 16, 16), 16)]
                srcb[slot][pl.ds(j * 16, 16)] = v >> 16
                dstb[slot][pl.ds(j * 16, 16)] = v & 0xFFFF

        def gather(c, slot):
            unpack(c, slot)
            pltpu.async_copy(h_hbm.at[srcb[slot]], rowb[slot], semg)

        def wait_gather(slot):
            pltpu.make_async_copy(h_hbm.at[pl.ds(0, K)], rowb[slot], semg).wait()

        for c in range(NBUF - 1):
            gather(c, c)

        base = sid * RPS
        for m in range(RPS // ZB):
            pltpu.async_copy(zbuf, agg.at[pl.ds(base + m * ZB, ZB)], semz)
        rem = RPS % ZB
        if rem:
            pltpu.async_copy(zbuf.at[pl.ds(0, rem)],
                             agg.at[pl.ds(base + RPS - rem, rem)], semz)
        for m in range(RPS // ZB):
            pltpu.make_async_copy(h_hbm.at[pl.ds(0, ZB)], zbuf, semz).wait()
        if rem:
            pltpu.make_async_copy(h_hbm.at[pl.ds(0, rem)],
                                  zbuf.at[pl.ds(0, rem)], semz).wait()
        plsc.subcore_barrier()

        def body(t, carry):
            c0 = NBUF * t
            for j in range(NBUF):
                wait_gather(j)

                @pl.when(c0 + j + NBUF - 1 < CHUNKS)
                def _():
                    gather(c0 + j + NBUF - 1, (j + NBUF - 1) % NBUF)

                pltpu.sync_copy(rowb[j], agg.at[dstb[j]], add=True)
            return carry

        lax.fori_loop(0, CHUNKS // NBUF, body, 0)
        plsc.subcore_barrier()

        @pl.when(sid < NSUB - 1)
        def _():
            pltpu.sync_copy(agg.at[pl.ds(sid * RPS, RPS)],
                            out_hbm.at[cid, pl.ds(sid * RPS, RPS)])

        @pl.when(sid == NSUB - 1)
        def _():
            tail = N - (NSUB - 1) * RPS
            pltpu.sync_copy(agg.at[pl.ds((NSUB - 1) * RPS, tail)],
                            out_hbm.at[cid, pl.ds((NSUB - 1) * RPS, tail)])

    return kern(h, packed_hbm)


BR = 2000


def _layer_body(h_ref, p0_ref, p1_ref, wr_ref, wt_ref, b_ref, o_ref, *, relu):
    agg = p0_ref[0] + p1_ref[0]
    o = (h_ref[...] + b_ref[...]
         + jnp.dot(agg, wr_ref[...], preferred_element_type=jnp.float32)
         + jnp.dot(h_ref[...], wt_ref[...], preferred_element_type=jnp.float32))
    if relu:
        o = jnp.maximum(o, 0.0)
    o_ref[...] = o


def _layer_tc(h, parts, W_rel, b_rel, W_root):
    return pl.pallas_call(
        functools.partial(_layer_body, relu=True),
        grid=(N // BR,),
        in_specs=[
            pl.BlockSpec((BR, D), lambda i: (i, 0)),
            pl.BlockSpec((1, BR, D), lambda i: (0, i, 0)),
            pl.BlockSpec((1, BR, D), lambda i: (1, i, 0)),
            pl.BlockSpec((D, D), lambda i: (0, 0)),
            pl.BlockSpec((D, D), lambda i: (0, 0)),
            pl.BlockSpec((1, D), lambda i: (0, 0)),
        ],
        out_specs=pl.BlockSpec((BR, D), lambda i: (i, 0)),
        out_shape=jax.ShapeDtypeStruct((N, D), jnp.float32),
    )(h, parts, parts, W_rel, W_root, b_rel.reshape(1, D))


def _final_body(h_ref, p0_ref, p1_ref, wr_ref, wt_ref, b_ref, wl_ref, bl_ref,
                o_ref):
    agg = p0_ref[0] + p1_ref[0]
    h5 = (h_ref[...] + b_ref[...]
          + jnp.dot(agg, wr_ref[...], preferred_element_type=jnp.float32)
          + jnp.dot(h_ref[...], wt_ref[...], preferred_element_type=jnp.float32))
    o_ref[...] = (jnp.dot(h5, wl_ref[...], preferred_element_type=jnp.float32)
                  + bl_ref[...])


def _final_tc(h, parts, W_rel, b_rel, W_root, W_lin, b_lin):
    return pl.pallas_call(
        _final_body,
        grid=(N // BR,),
        in_specs=[
            pl.BlockSpec((BR, D), lambda i: (i, 0)),
            pl.BlockSpec((1, BR, D), lambda i: (0, i, 0)),
            pl.BlockSpec((1, BR, D), lambda i: (1, i, 0)),
            pl.BlockSpec((D, D), lambda i: (0, 0)),
            pl.BlockSpec((D, D), lambda i: (0, 0)),
            pl.BlockSpec((1, D), lambda i: (0, 0)),
            pl.BlockSpec((D, OUT), lambda i: (0, 0)),
            pl.BlockSpec((1, OUT), lambda i: (0, 0)),
        ],
        out_specs=pl.BlockSpec((BR, OUT), lambda i: (i, 0)),
        out_shape=jax.ShapeDtypeStruct((N, OUT), jnp.float32),
    )(h, parts, parts, W_rel, W_root, b_rel.reshape(1, D),
      W_lin, b_lin.reshape(1, OUT))


def kernel(x, edge_index,
           W_rel1, b_rel1, W_root1, W_rel2, b_rel2, W_root2,
           W_rel3, b_rel3, W_root3, W_rel4, b_rel4, W_root4,
           W_rel5, b_rel5, W_root5, W_lin, b_lin):
    src, dst = edge_index[0], edge_index[1]
    pad = jnp.arange(E_PAD - E, dtype=jnp.int32)
    src_p = jnp.concatenate([src, (pad * 7919) % N])
    dst_p = jnp.concatenate([dst, N + pad % JUNK])
    packed = ((src_p << 16) | dst_p).reshape(NW, CHUNKS * K)
    layers = [(W_rel1, b_rel1, W_root1), (W_rel2, b_rel2, W_root2),
              (W_rel3, b_rel3, W_root3), (W_rel4, b_rel4, W_root4)]
    h = x
    for (Wr, br, Wt) in layers:
        parts = _segment_sum_partials(h, packed)
        h = _layer_tc(h, parts, Wr, br, Wt)
    parts = _segment_sum_partials(h, packed)
    return _final_tc(h, parts, W_rel5, b_rel5, W_root5, W_lin, b_lin)
